# baseline (speedup 1.0000x reference)
import functools

import jax
import jax.numpy as jnp
from jax import lax
from jax.experimental import pallas as pl
from jax.experimental.pallas import tpu as pltpu
from jax.experimental.pallas import tpu_sc as plsc

_SMOOTH = 0.1
_RB = 32
_LANE = 128
_R0 = 0
_NW = 32
_TILES = 11
_CW = _TILES * _LANE
_NCH = 71
_CFULL = _NCH * _CW


def _tc_kernel(tgt_smem, *rest, num_classes):
    strips = rest[: _RB]
    out2_ref = rest[_RB]

    for j in range(_RB):
        out2_ref[pl.ds(j, 1), :] = strips[j][j % 8, :].reshape(1, _LANE)


def _strip_spec(j, base):
    def index_map(i, tref):
        r = base + i * _RB + j
        return (r // 8, tref[r] // _LANE)

    return pl.BlockSpec((8, _LANE), index_map)


def _sc_body(pred_hbm, m_out, s_out, e_out, buf, res, sem):
    wid = lax.axis_index("s") * 2 + lax.axis_index("c")
    neg_inf = jnp.full((16,), -jnp.inf, jnp.float32)
    zero = jnp.zeros((16,), jnp.float32)

    for k in range(4):
        r0 = _R0 + (4 * wid + k) * 8

        def _copy(c, slot):
            return pltpu.make_async_copy(
                pred_hbm.at[pl.ds(r0, 8), pl.ds(c * _CW, _CW)],
                buf.at[slot],
                sem.at[slot],
            )

        _copy(0, 0).start()

        def chunk_body(c, carry):
            ms, ss, es = carry
            b = lax.rem(c, 2)

            @pl.when(c + 1 < _NCH)
            def _():
                _copy(c + 1, lax.rem(c + 1, 2)).start()

            _copy(c, b).wait()

            def p1(t, cr):
                cm, cs = cr
                ncm, ncs = [], []
                for r in range(8):
                    acc_m, acc_s = cm[r], cs[r]
                    for v in range(8):
                        xv = buf[b, r, pl.ds(t * _LANE + v * 16, 16)]
                        acc_m = jnp.maximum(acc_m, xv)
                        acc_s = acc_s + xv
                    ncm.append(acc_m)
                    ncs.append(acc_s)
                return tuple(ncm), tuple(ncs)

            cm, cs = lax.fori_loop(
                0, _TILES, p1, ((neg_inf,) * 8, (zero,) * 8)
            )

            def p2(t, ce):
                nce = []
                for r in range(8):
                    acc_e = ce[r]
                    for v in range(8):
                        xv = buf[b, r, pl.ds(t * _LANE + v * 16, 16)]
                        acc_e = acc_e + jnp.exp(xv - cm[r])
                    nce.append(acc_e)
                return tuple(nce)

            ec = lax.fori_loop(0, _TILES, p2, (zero,) * 8)

            nms, nss, nes = [], [], []
            for r in range(8):
                mn = jnp.maximum(ms[r], cm[r])
                e_new = es[r] * jnp.exp(ms[r] - mn) + ec[r] * jnp.exp(cm[r] - mn)
                nms.append(mn)
                nss.append(ss[r] + cs[r])
                nes.append(e_new)
            return tuple(nms), tuple(nss), tuple(nes)

        ms, ss, es = lax.fori_loop(
            0, _NCH, chunk_body,
            ((neg_inf,) * 8, (zero,) * 8, (zero,) * 8),
        )

        base = (32 * wid + 8 * k) * 16
        for r in range(8):
            res[pl.ds(16 * r, 16)] = ms[r]
        pltpu.sync_copy(res, m_out.at[pl.ds(base, 128)])
        for r in range(8):
            res[pl.ds(16 * r, 16)] = ss[r]
        pltpu.sync_copy(res, s_out.at[pl.ds(base, 128)])
        for r in range(8):
            res[pl.ds(16 * r, 16)] = es[r]
        pltpu.sync_copy(res, e_out.at[pl.ds(base, 128)])


def _sc_partials(pred):
    n_sc = pred.shape[0] - _R0
    out = jax.ShapeDtypeStruct((n_sc * 16,), jnp.float32)
    k = functools.partial(
        pl.kernel,
        out_type=[out, out, out],
        mesh=plsc.VectorSubcoreMesh(core_axis_name="c", subcore_axis_name="s"),
        scratch_types=[
            pltpu.VMEM((2, 8, _CW), jnp.float32),
            pltpu.VMEM((128,), jnp.float32),
            pltpu.SemaphoreType.DMA((2,)),
        ],
        compiler_params=pltpu.CompilerParams(use_tc_tiling_on_sc=True),
    )(_sc_body)
    return k(pred)


def _combine_kernel(strips2_ref, tgt_ref, ml_ref, sl_ref, el_ref,
                    tail_ref, out_ref, *, num_classes, batch):
    n_sc = batch - _R0
    tail_w = num_classes - _CFULL

    lane = lax.broadcasted_iota(jnp.int32, (n_sc, _LANE), 1)
    tail = tail_ref[...]
    tmask = lane < tail_w
    neg_inf = jnp.float32(-jnp.inf)
    m_t = jnp.max(jnp.where(tmask, tail, neg_inf), axis=1, keepdims=True)
    s_t = jnp.sum(jnp.where(tmask, tail, 0.0), axis=1, keepdims=True)
    e_t = jnp.sum(jnp.where(tmask, jnp.exp(tail - m_t), 0.0), axis=1,
                  keepdims=True)

    ml = ml_ref[...]
    m_s = jnp.max(ml, axis=1, keepdims=True)
    m = jnp.maximum(m_s, m_t)
    e = (jnp.sum(el_ref[...] * jnp.exp(ml - m), axis=1, keepdims=True)
         + e_t * jnp.exp(m_t - m))
    s_sum = jnp.sum(sl_ref[...], axis=1, keepdims=True) + s_t
    lse = jnp.log(e)

    off = lax.rem(tgt_ref[...], _LANE)
    pt = jnp.sum(jnp.where(lane == off, strips2_ref[...], 0.0), axis=1,
                 keepdims=True)

    eps = _SMOOTH / (num_classes - 1)
    coef = 1.0 - _SMOOTH - eps
    row_loss = -(
        eps * (s_sum - num_classes * (m + lse)) + coef * (pt - m - lse)
    )
    total = jnp.sum(row_loss) / batch
    out_ref[...] = total.reshape(1, 1)


def kernel(pred, target):
    batch, num_classes = pred.shape
    tgt = target.astype(jnp.int32)
    n_sc = batch - _R0
    grid = batch // _RB

    m_l, s_l, e_l = _sc_partials(pred)

    grid_spec = pltpu.PrefetchScalarGridSpec(
        num_scalar_prefetch=1,
        grid=(grid,),
        in_specs=[
            *[_strip_spec(j, 0) for j in range(_RB)],
        ],
        out_specs=pl.BlockSpec((_RB, _LANE), lambda i, tref: (i, 0)),
    )
    strips2 = pl.pallas_call(
        functools.partial(_tc_kernel, num_classes=num_classes),
        grid_spec=grid_spec,
        out_shape=jax.ShapeDtypeStruct((n_sc, _LANE), jnp.float32),
    )(tgt, *([pred] * _RB))

    loss = pl.pallas_call(
        functools.partial(_combine_kernel, num_classes=num_classes,
                          batch=batch),
        grid=(1,),
        in_specs=[
            pl.BlockSpec((n_sc, _LANE), lambda i: (0, 0)),
            pl.BlockSpec((n_sc, 1), lambda i: (0, 0)),
            pl.BlockSpec((n_sc, 16), lambda i: (0, 0)),
            pl.BlockSpec((n_sc, 16), lambda i: (0, 0)),
            pl.BlockSpec((n_sc, 16), lambda i: (0, 0)),
            pl.BlockSpec((n_sc, _LANE), lambda i: (0, num_classes // _LANE)),
        ],
        out_specs=pl.BlockSpec((1, 1), lambda i: (0, 0)),
        out_shape=jax.ShapeDtypeStruct((1, 1), jnp.float32),
    )(
        strips2,
        tgt.reshape(n_sc, 1),
        m_l.reshape(n_sc, 16),
        s_l.reshape(n_sc, 16),
        e_l.reshape(n_sc, 16),
        pred,
    )
    return loss[0, 0]


# speedup vs baseline: 1.2681x; 1.2681x over previous
import functools

import jax
import jax.numpy as jnp
from jax import lax
from jax.experimental import pallas as pl
from jax.experimental.pallas import tpu as pltpu
from jax.experimental.pallas import tpu_sc as plsc

_SMOOTH = 0.1
_RB = 32
_LANE = 128
_R0 = 512
_NW = 32
_TILES = 11
_CW = _TILES * _LANE
_NCH = 71
_CFULL = _NCH * _CW


def _tc_kernel(tgt_smem, pred_ref, *rest, num_classes):
    strips = rest[: _RB]
    scstrips = rest[_RB: 2 * _RB]
    out_ref, out2_ref = rest[2 * _RB], rest[2 * _RB + 1]
    i = pl.program_id(0)

    x = pred_ref[...]
    m = jnp.max(x, axis=1, keepdims=True)
    s_sum = jnp.sum(x, axis=1, keepdims=True)
    e_sum = jnp.sum(jnp.exp(x - m), axis=1, keepdims=True)
    lse = jnp.log(e_sum)

    eps = _SMOOTH / (num_classes - 1)
    coef = 1.0 - _SMOOTH - eps
    vec_part = -(eps * (s_sum - num_classes * (m + lse)) + coef * (-m - lse))

    lane = lax.broadcasted_iota(jnp.int32, (1, _LANE), 1)
    pt_total = 0.0
    for j in range(_RB):
        t = tgt_smem[i * _RB + j]
        off = lax.rem(t, _LANE)
        row = strips[j][j % 8, :].reshape(1, _LANE)
        pt_total += jnp.sum(jnp.where(lane == off, row, 0.0))

    block_sum = jnp.sum(vec_part) - coef * pt_total

    for j in range(_RB):
        out2_ref[pl.ds(j, 1), :] = scstrips[j][j % 8, :].reshape(1, _LANE)

    @pl.when(i == 0)
    def _():
        out_ref[...] = jnp.zeros((1, 1), jnp.float32)

    out_ref[...] += block_sum.reshape(1, 1)


def _strip_spec(j, base):
    def index_map(i, tref):
        r = base + i * _RB + j
        return (r // 8, tref[r] // _LANE)

    return pl.BlockSpec((8, _LANE), index_map)


def _sc_body(pred_hbm, m_out, s_out, e_out, buf, res, sem):
    wid = lax.axis_index("s") * 2 + lax.axis_index("c")
    neg_inf = jnp.full((16,), -jnp.inf, jnp.float32)
    zero = jnp.zeros((16,), jnp.float32)

    for k in range(2):
        r0 = _R0 + (2 * wid + k) * 8

        def _copy(c, slot):
            return pltpu.make_async_copy(
                pred_hbm.at[pl.ds(r0, 8), pl.ds(c * _CW, _CW)],
                buf.at[slot],
                sem.at[slot],
            )

        _copy(0, 0).start()

        def chunk_body(c, carry):
            ms, ss, es = carry
            b = lax.rem(c, 2)

            @pl.when(c + 1 < _NCH)
            def _():
                _copy(c + 1, lax.rem(c + 1, 2)).start()

            _copy(c, b).wait()

            def p1(t, cr):
                cm, cs = cr
                ncm, ncs = [], []
                for r in range(8):
                    acc_m, acc_s = cm[r], cs[r]
                    for v in range(8):
                        xv = buf[b, r, pl.ds(t * _LANE + v * 16, 16)]
                        acc_m = jnp.maximum(acc_m, xv)
                        acc_s = acc_s + xv
                    ncm.append(acc_m)
                    ncs.append(acc_s)
                return tuple(ncm), tuple(ncs)

            cm, cs = lax.fori_loop(
                0, _TILES, p1, ((neg_inf,) * 8, (zero,) * 8)
            )

            def p2(t, ce):
                nce = []
                for r in range(8):
                    acc_e = ce[r]
                    for v in range(8):
                        xv = buf[b, r, pl.ds(t * _LANE + v * 16, 16)]
                        acc_e = acc_e + jnp.exp(xv - cm[r])
                    nce.append(acc_e)
                return tuple(nce)

            ec = lax.fori_loop(0, _TILES, p2, (zero,) * 8)

            nms, nss, nes = [], [], []
            for r in range(8):
                mn = jnp.maximum(ms[r], cm[r])
                e_new = es[r] * jnp.exp(ms[r] - mn) + ec[r] * jnp.exp(cm[r] - mn)
                nms.append(mn)
                nss.append(ss[r] + cs[r])
                nes.append(e_new)
            return tuple(nms), tuple(nss), tuple(nes)

        ms, ss, es = lax.fori_loop(
            0, _NCH, chunk_body,
            ((neg_inf,) * 8, (zero,) * 8, (zero,) * 8),
        )

        base = (16 * wid + 8 * k) * 16
        for r in range(8):
            res[pl.ds(16 * r, 16)] = ms[r]
        pltpu.sync_copy(res, m_out.at[pl.ds(base, 128)])
        for r in range(8):
            res[pl.ds(16 * r, 16)] = ss[r]
        pltpu.sync_copy(res, s_out.at[pl.ds(base, 128)])
        for r in range(8):
            res[pl.ds(16 * r, 16)] = es[r]
        pltpu.sync_copy(res, e_out.at[pl.ds(base, 128)])


def _sc_partials(pred):
    n_sc = pred.shape[0] - _R0
    out = jax.ShapeDtypeStruct((n_sc * 16,), jnp.float32)
    k = functools.partial(
        pl.kernel,
        out_type=[out, out, out],
        mesh=plsc.VectorSubcoreMesh(core_axis_name="c", subcore_axis_name="s"),
        scratch_types=[
            pltpu.VMEM((2, 8, _CW), jnp.float32),
            pltpu.VMEM((128,), jnp.float32),
            pltpu.SemaphoreType.DMA((2,)),
        ],
        compiler_params=pltpu.CompilerParams(use_tc_tiling_on_sc=True),
    )(_sc_body)
    return k(pred)


def _combine_kernel(tc_sum_ref, strips2_ref, tgt_ref, ml_ref, sl_ref, el_ref,
                    tail_ref, out_ref, *, num_classes, batch):
    n_sc = batch - _R0
    tail_w = num_classes - _CFULL

    lane = lax.broadcasted_iota(jnp.int32, (n_sc, _LANE), 1)
    tail = tail_ref[...]
    tmask = lane < tail_w
    neg_inf = jnp.float32(-jnp.inf)
    m_t = jnp.max(jnp.where(tmask, tail, neg_inf), axis=1, keepdims=True)
    s_t = jnp.sum(jnp.where(tmask, tail, 0.0), axis=1, keepdims=True)
    e_t = jnp.sum(jnp.where(tmask, jnp.exp(tail - m_t), 0.0), axis=1,
                  keepdims=True)

    ml = ml_ref[...]
    m_s = jnp.max(ml, axis=1, keepdims=True)
    m = jnp.maximum(m_s, m_t)
    e = (jnp.sum(el_ref[...] * jnp.exp(ml - m), axis=1, keepdims=True)
         + e_t * jnp.exp(m_t - m))
    s_sum = jnp.sum(sl_ref[...], axis=1, keepdims=True) + s_t
    lse = jnp.log(e)

    off = lax.rem(tgt_ref[...], _LANE)
    pt = jnp.sum(jnp.where(lane == off, strips2_ref[...], 0.0), axis=1,
                 keepdims=True)

    eps = _SMOOTH / (num_classes - 1)
    coef = 1.0 - _SMOOTH - eps
    row_loss = -(
        eps * (s_sum - num_classes * (m + lse)) + coef * (pt - m - lse)
    )
    total = (jnp.sum(row_loss) + tc_sum_ref[0, 0]) / batch
    out_ref[...] = total.reshape(1, 1)


def kernel(pred, target):
    batch, num_classes = pred.shape
    tgt = target.astype(jnp.int32)
    n_sc = batch - _R0
    grid = _R0 // _RB

    grid_spec = pltpu.PrefetchScalarGridSpec(
        num_scalar_prefetch=1,
        grid=(grid,),
        in_specs=[
            pl.BlockSpec((_RB, num_classes), lambda i, tref: (i, 0)),
            *[_strip_spec(j, 0) for j in range(_RB)],
            *[_strip_spec(j, _R0) for j in range(_RB)],
        ],
        out_specs=[
            pl.BlockSpec((1, 1), lambda i, tref: (0, 0)),
            pl.BlockSpec((_RB, _LANE), lambda i, tref: (i, 0)),
        ],
    )
    tc_sum, strips2 = pl.pallas_call(
        functools.partial(_tc_kernel, num_classes=num_classes),
        grid_spec=grid_spec,
        out_shape=[
            jax.ShapeDtypeStruct((1, 1), jnp.float32),
            jax.ShapeDtypeStruct((n_sc, _LANE), jnp.float32),
        ],
    )(tgt, pred, *([pred] * (2 * _RB)))

    m_l, s_l, e_l = _sc_partials(pred)

    loss = pl.pallas_call(
        functools.partial(_combine_kernel, num_classes=num_classes,
                          batch=batch),
        grid=(1,),
        in_specs=[
            pl.BlockSpec((1, 1), lambda i: (0, 0)),
            pl.BlockSpec((n_sc, _LANE), lambda i: (0, 0)),
            pl.BlockSpec((n_sc, 1), lambda i: (0, 0)),
            pl.BlockSpec((n_sc, 16), lambda i: (0, 0)),
            pl.BlockSpec((n_sc, 16), lambda i: (0, 0)),
            pl.BlockSpec((n_sc, 16), lambda i: (0, 0)),
            pl.BlockSpec((n_sc, _LANE),
                         lambda i: (_R0 // (1024 - _R0), num_classes // _LANE)),
        ],
        out_specs=pl.BlockSpec((1, 1), lambda i: (0, 0)),
        out_shape=jax.ShapeDtypeStruct((1, 1), jnp.float32),
    )(
        tc_sum,
        strips2,
        tgt[_R0:].reshape(n_sc, 1),
        m_l.reshape(n_sc, 16),
        s_l.reshape(n_sc, 16),
        e_l.reshape(n_sc, 16),
        pred,
    )
    return loss[0, 0]
